# baseline (speedup 1.0000x reference)
import functools

import jax
import jax.numpy as jnp
from jax import lax
from jax.experimental import pallas as pl
from jax.experimental.pallas import tpu as pltpu
from jax.experimental.pallas import tpu_sc as plsc

_NC = 2
_NS = 16
_NW = _NC * _NS
_LN = 128


def _sc_mesh():
    return plsc.VectorSubcoreMesh(
        core_axis_name="c", subcore_axis_name="s", num_cores=_NC, num_subcores=_NS
    )


def _rsqrt16(d):
    i = lax.bitcast_convert_type(d, jnp.int32)
    y = lax.bitcast_convert_type(jnp.int32(0x5F3759DF) - (i >> 1), jnp.float32)
    for _ in range(3):
        y = y * (1.5 - 0.5 * d * y * y)
    return y


def _scale_rows(xw_hbm, dv, buf, y_s, base):
    blk = buf.at[0].at[pl.ds(0, _LN)]
    for k in range(5):
        off = base + _LN * k
        pltpu.sync_copy(xw_hbm.at[pl.ds(off, _LN)], blk)

        def sgrp(i, carry, k=k):
            dvec = dv[pl.ds(_LN * k + 16 * i, 16)]
            for u in range(16):
                s = jnp.broadcast_to(lax.slice(dvec, (u,), (u + 1,)), (16,))
                for q in range(4):
                    blk[16 * i + u, pl.ds(16 * q, 16)] = (
                        blk[16 * i + u, pl.ds(16 * q, 16)] * s)
            return carry

        lax.fori_loop(0, _LN // 16, sgrp, 0)
        pltpu.sync_copy(blk, y_s.at[pl.ds(off, _LN)])


def _zero_acc(y_hbm, acc, base, npad):
    zsrc = y_hbm.at[pl.ds(npad - _LN, _LN)]
    for k in range(5):
        pltpu.sync_copy(zsrc, acc.at[pl.ds(base + _LN * k, _LN)])


def _prime_gathers(y_ref, src_v, buf, gsems):
    gw = 2 * _LN
    for q in range(4):
        pltpu.async_copy(y_ref.at[src_v.at[pl.ds(gw * q, gw)]], buf.at[q], gsems[q])


def _edge_pipeline(y_ref, acc, src_v, dst_v, buf, gsems, ssems, ch):
    nit = ch // 8
    gw = 2 * _LN

    def fire_g(q, grp):
        pltpu.async_copy(y_ref.at[src_v.at[pl.ds(gw * grp, gw)]], buf.at[q], gsems[q])

    def drain_g(q, grp):
        pltpu.make_async_copy(y_ref.at[src_v.at[pl.ds(gw * grp, gw)]], buf.at[q], gsems[q]).wait()

    def fire_s(q, grp):
        pltpu.async_copy(buf.at[q], acc.at[dst_v.at[pl.ds(gw * grp, gw)]], ssems[q], add=True)

    def drain_s(q, grp):
        pltpu.make_async_copy(buf.at[q], acc.at[dst_v.at[pl.ds(gw * grp, gw)]], ssems[q]).wait()

    def body(gg, carry):
        g0 = 4 * gg
        for q in range(4):
            drain_g(q, g0 + q)
            fire_s(q, g0 + q)
        for q in range(4):
            drain_s(q, g0 + q)

            @pl.when(gg < nit - 1)
            def _(q=q):
                fire_g(q, g0 + q + 4)

        return carry

    lax.fori_loop(0, nit, body, 0)


def _make_deg_kernel(npad, ch):
    rpt = npad // _NS

    @functools.partial(
        pl.kernel,
        out_type=jax.ShapeDtypeStruct((_NC, npad), jnp.float32),
        mesh=_sc_mesh(),
        compiler_params=pltpu.CompilerParams(use_tc_tiling_on_sc=False),
        scratch_types=[
            pltpu.VMEM((ch * _LN,), jnp.int32),
            pltpu.VMEM((ch * _LN,), jnp.float32),
            pltpu.VMEM((rpt,), jnp.float32),
            pltpu.VMEM_SHARED((npad,), jnp.float32),
            pltpu.SemaphoreType.DMA,
        ],
    )
    def deg_kernel(dstg_hbm, out_hbm, dst_v, ones_v, zero_v, acc, ssem):
        cid = lax.axis_index("c")
        sid = lax.axis_index("s")
        wid = sid * _NC + cid
        pltpu.sync_copy(dstg_hbm.at[wid], dst_v)

        def fill(j, carry):
            for u in range(8):
                ones_v[pl.ds((8 * j + u) * 16, 16)] = jnp.full((16,), 1.0, jnp.float32)
            return carry

        lax.fori_loop(0, ch * _LN // 128, fill, 0)

        def fillz(j, carry):
            for u in range(8):
                zero_v[pl.ds((8 * j + u) * 16, 16)] = jnp.zeros((16,), jnp.float32)
            return carry

        lax.fori_loop(0, rpt // 128, fillz, 0)
        pltpu.sync_copy(zero_v, acc.at[pl.ds(sid * rpt, rpt)])
        plsc.subcore_barrier()
        pltpu.async_copy(ones_v, acc.at[dst_v], ssem, add=True).wait()
        plsc.subcore_barrier()
        pltpu.sync_copy(acc.at[pl.ds(sid * rpt, rpt)], out_hbm.at[cid, pl.ds(sid * rpt, rpt)])

    return deg_kernel


def _make_msg_kernel(npad, ch):
    rpt = npad // _NS

    @functools.partial(
        pl.kernel,
        out_type=jax.ShapeDtypeStruct((_NC, npad, 64), jnp.float32),
        mesh=_sc_mesh(),
        compiler_params=pltpu.CompilerParams(use_tc_tiling_on_sc=False),
        scratch_types=[
            pltpu.VMEM((ch * _LN,), jnp.int32),
            pltpu.VMEM((ch * _LN,), jnp.int32),
            pltpu.VMEM((4, 2 * _LN, 64), jnp.float32),
            pltpu.VMEM_SHARED((npad, 64), jnp.float32),
            [pltpu.SemaphoreType.DMA] * 4,
            [pltpu.SemaphoreType.DMA] * 4,
        ],
    )
    def msg2(y_hbm, srcg_hbm, dstg_hbm, s_out,
             dst_v, src_v, buf, acc, gsems, ssems):
        cid = lax.axis_index("c")
        sid = lax.axis_index("s")
        wid = sid * _NC + cid
        base = sid * rpt

        pltpu.sync_copy(srcg_hbm.at[wid], src_v)
        _prime_gathers(y_hbm, src_v, buf, gsems)
        pltpu.sync_copy(dstg_hbm.at[wid], dst_v)
        _zero_acc(y_hbm, acc, base, npad)
        plsc.subcore_barrier()
        _edge_pipeline(y_hbm, acc, src_v, dst_v, buf, gsems, ssems, ch)
        plsc.subcore_barrier()
        pltpu.sync_copy(acc.at[pl.ds(base, rpt)], s_out.at[cid, pl.ds(base, rpt)])

    return msg2


def _tca_body(xe_ref, xo_ref, w1l_ref, w1r_ref, xw_ref, *, nh):
    xw = (jnp.dot(xe_ref[...], w1l_ref[...], preferred_element_type=jnp.float32)
          + jnp.dot(xo_ref[...], w1r_ref[...], preferred_element_type=jnp.float32))
    xw_ref[:nh, :] = xw
    xw_ref[nh:, :] = jnp.zeros_like(xw_ref[nh:, :])


def _tcb_body(xw1_ref, deg_ref, y1_ref):
    y1_ref[...] = xw1_ref[...] * _dinv_paired(deg_ref)


def _dinv_paired(deg_ref):
    d = deg_ref[...]
    dinv2 = lax.rsqrt(1.0 + d[0] + d[1])
    a = jnp.broadcast_to(dinv2[:, :1], (dinv2.shape[0], 64))
    b = jnp.broadcast_to(dinv2[:, 1:2], (dinv2.shape[0], 64))
    return jnp.concatenate([a, b], axis=1)


def _tc2_body(y1_ref, s_ref, deg_ref, b1_ref, w2_ref, y2_ref, *, nh):
    dp = _dinv_paired(deg_ref)
    s = s_ref[...]
    h = jnp.maximum(dp * (y1_ref[...] + s[0] + s[1]) + b1_ref[...], 0.0)
    rows = lax.broadcasted_iota(jnp.int32, h.shape, 0)
    h = jnp.where(rows < nh, h, 0.0)
    y2_ref[...] = jnp.dot(h, w2_ref[...], preferred_element_type=jnp.float32) * dp


def _tc3_body(y2_ref, s_ref, deg_ref, b2_ref, out_ref, *, nh):
    dp = _dinv_paired(deg_ref)
    s = s_ref[...]
    tot = dp * (y2_ref[...] + s[0] + s[1]) + b2_ref[...]
    out_ref[...] = tot[:nh, :]


def kernel(x, edge_index, W1, b1, W2, b2):
    n, fin = x.shape
    fh = W1.shape[1]
    e = edge_index.shape[1]
    f32 = jnp.float32

    npad = ((n + 16 + 511) // 512) * 512
    ch = -(-e // (_NW * _LN))
    ch = ((ch + 7) // 8) * 8
    epad = _NW * _LN * ch

    src = edge_index[0].astype(jnp.int32)
    dst = edge_index[1].astype(jnp.int32)
    pad_idx = n + (jnp.arange(epad - e, dtype=jnp.int32) % (npad - n))
    srcg = jnp.concatenate([src, pad_idx]).reshape(_NW, ch * _LN)
    dstg = jnp.concatenate([dst, pad_idx]).reshape(_NW, ch * _LN)

    x_even = x[0::2]
    x_odd = x[1::2]
    w1l = jnp.zeros((fin, 128), f32).at[:, :fh].set(W1)
    w1r = jnp.zeros((fin, 128), f32).at[:, fh:].set(W1)
    w2bd = (jnp.zeros((128, 128), f32)
            .at[:fh, :fh].set(W2).at[fh:, fh:].set(W2))
    b1p = jnp.concatenate([b1, b1]).reshape(1, 128)
    b2p = jnp.concatenate([b2, b2]).reshape(1, 128)

    nh = n // 2
    nph = npad // 2

    deg_parts = _make_deg_kernel(npad, ch)(dstg)
    deg2 = deg_parts.reshape(_NC, nph, 2)

    xw1 = pl.pallas_call(
        functools.partial(_tca_body, nh=nh),
        out_shape=jax.ShapeDtypeStruct((nph, 128), f32),
    )(x_even, x_odd, w1l, w1r)

    y1 = pl.pallas_call(
        _tcb_body,
        out_shape=jax.ShapeDtypeStruct((nph, 128), f32),
    )(xw1, deg2)

    msg = _make_msg_kernel(npad, ch)
    s1 = msg(y1.reshape(npad, 64), srcg, dstg)
    s1p = s1.reshape(_NC, nph, 128)

    y2 = pl.pallas_call(
        functools.partial(_tc2_body, nh=nh),
        out_shape=jax.ShapeDtypeStruct((nph, 128), f32),
    )(y1, s1p, deg2, b1p, w2bd)

    s2 = msg(y2.reshape(npad, 64), srcg, dstg)
    s2p = s2.reshape(_NC, nph, 128)

    out = pl.pallas_call(
        functools.partial(_tc3_body, nh=nh),
        out_shape=jax.ShapeDtypeStruct((nh, 128), f32),
    )(y2, s2p, deg2, b2p)
    return out.reshape(n, fh)


# speedup vs baseline: 1.0482x; 1.0482x over previous
import functools

import jax
import jax.numpy as jnp
from jax import lax
from jax.experimental import pallas as pl
from jax.experimental.pallas import tpu as pltpu
from jax.experimental.pallas import tpu_sc as plsc

_NC = 2
_NS = 16
_NW = _NC * _NS
_LN = 128


def _sc_mesh():
    return plsc.VectorSubcoreMesh(
        core_axis_name="c", subcore_axis_name="s", num_cores=_NC, num_subcores=_NS
    )


def _rsqrt16(d):
    i = lax.bitcast_convert_type(d, jnp.int32)
    y = lax.bitcast_convert_type(jnp.int32(0x5F3759DF) - (i >> 1), jnp.float32)
    for _ in range(3):
        y = y * (1.5 - 0.5 * d * y * y)
    return y


def _scale_rows(xw_hbm, dv, buf, y_s, base):
    blk = buf.at[0].at[pl.ds(0, _LN)]
    for k in range(5):
        off = base + _LN * k
        pltpu.sync_copy(xw_hbm.at[pl.ds(off, _LN)], blk)

        def sgrp(i, carry, k=k):
            dvec = dv[pl.ds(_LN * k + 16 * i, 16)]
            for u in range(16):
                s = jnp.broadcast_to(lax.slice(dvec, (u,), (u + 1,)), (16,))
                for q in range(4):
                    blk[16 * i + u, pl.ds(16 * q, 16)] = (
                        blk[16 * i + u, pl.ds(16 * q, 16)] * s)
            return carry

        lax.fori_loop(0, _LN // 16, sgrp, 0)
        pltpu.sync_copy(blk, y_s.at[pl.ds(off, _LN)])


def _zero_acc(buf, acc, base):
    blk = buf.at[0].at[pl.ds(0, _LN)]

    def zrow(j, carry):
        for q in range(4):
            blk[j, pl.ds(16 * q, 16)] = jnp.zeros((16,), jnp.float32)
        return carry

    lax.fori_loop(0, _LN, zrow, 0)
    for k in range(5):
        pltpu.sync_copy(blk, acc.at[pl.ds(base + _LN * k, _LN)])


def _edge_pipeline(y_ref, acc, src_v, dst_v, buf, gsems, ssems, ch):
    nit = ch // 8
    gw = 2 * _LN

    def fire_g(q, grp):
        pltpu.async_copy(y_ref.at[src_v.at[pl.ds(gw * grp, gw)]], buf.at[q], gsems[q])

    def drain_g(q, grp):
        pltpu.make_async_copy(y_ref.at[src_v.at[pl.ds(gw * grp, gw)]], buf.at[q], gsems[q]).wait()

    def fire_s(q, grp):
        pltpu.async_copy(buf.at[q], acc.at[dst_v.at[pl.ds(gw * grp, gw)]], ssems[q], add=True)

    def drain_s(q, grp):
        pltpu.make_async_copy(buf.at[q], acc.at[dst_v.at[pl.ds(gw * grp, gw)]], ssems[q]).wait()

    for q in range(4):
        fire_g(q, q)

    def body(gg, carry):
        g0 = 4 * gg
        for q in range(4):
            drain_g(q, g0 + q)
            fire_s(q, g0 + q)
        for q in range(4):
            drain_s(q, g0 + q)

            @pl.when(gg < nit - 1)
            def _(q=q):
                fire_g(q, g0 + q + 4)

        return carry

    lax.fori_loop(0, nit, body, 0)


def _make_deg_kernel(npad, ch):
    rpt = npad // _NS

    @functools.partial(
        pl.kernel,
        out_type=jax.ShapeDtypeStruct((_NC, npad), jnp.float32),
        mesh=_sc_mesh(),
        compiler_params=pltpu.CompilerParams(use_tc_tiling_on_sc=False),
        scratch_types=[
            pltpu.VMEM((ch * _LN,), jnp.int32),
            pltpu.VMEM((ch * _LN,), jnp.float32),
            pltpu.VMEM((rpt,), jnp.float32),
            pltpu.VMEM_SHARED((npad,), jnp.float32),
            pltpu.SemaphoreType.DMA,
        ],
    )
    def deg_kernel(dstg_hbm, out_hbm, dst_v, ones_v, zero_v, acc, ssem):
        cid = lax.axis_index("c")
        sid = lax.axis_index("s")
        wid = sid * _NC + cid
        pltpu.sync_copy(dstg_hbm.at[wid], dst_v)

        def fill(j, carry):
            for u in range(8):
                ones_v[pl.ds((8 * j + u) * 16, 16)] = jnp.full((16,), 1.0, jnp.float32)
            return carry

        lax.fori_loop(0, ch * _LN // 128, fill, 0)

        def fillz(j, carry):
            for u in range(8):
                zero_v[pl.ds((8 * j + u) * 16, 16)] = jnp.zeros((16,), jnp.float32)
            return carry

        lax.fori_loop(0, rpt // 128, fillz, 0)
        pltpu.sync_copy(zero_v, acc.at[pl.ds(sid * rpt, rpt)])
        plsc.subcore_barrier()
        pltpu.async_copy(ones_v, acc.at[dst_v], ssem, add=True).wait()
        plsc.subcore_barrier()
        pltpu.sync_copy(acc.at[pl.ds(sid * rpt, rpt)], out_hbm.at[cid, pl.ds(sid * rpt, rpt)])

    return deg_kernel


def _make_msg_kernel(npad, ch):
    rpt = npad // _NS

    @functools.partial(
        pl.kernel,
        out_type=jax.ShapeDtypeStruct((_NC, npad, 64), jnp.float32),
        mesh=_sc_mesh(),
        compiler_params=pltpu.CompilerParams(use_tc_tiling_on_sc=False),
        scratch_types=[
            pltpu.VMEM((ch * _LN,), jnp.int32),
            pltpu.VMEM((ch * _LN,), jnp.int32),
            pltpu.VMEM((4, 2 * _LN, 64), jnp.float32),
            pltpu.VMEM_SHARED((npad, 64), jnp.float32),
            [pltpu.SemaphoreType.DMA] * 4,
            [pltpu.SemaphoreType.DMA] * 4,
        ],
    )
    def msg2(y_hbm, srcg_hbm, dstg_hbm, s_out,
             dst_v, src_v, buf, acc, gsems, ssems):
        cid = lax.axis_index("c")
        sid = lax.axis_index("s")
        wid = sid * _NC + cid
        base = sid * rpt

        pltpu.sync_copy(dstg_hbm.at[wid], dst_v)
        pltpu.sync_copy(srcg_hbm.at[wid], src_v)
        _zero_acc(buf, acc, base)
        plsc.subcore_barrier()
        _edge_pipeline(y_hbm, acc, src_v, dst_v, buf, gsems, ssems, ch)
        plsc.subcore_barrier()
        pltpu.sync_copy(acc.at[pl.ds(base, rpt)], s_out.at[cid, pl.ds(base, rpt)])

    return msg2


def _tca_body(xe_ref, xo_ref, w1l_ref, w1r_ref, xw_ref, *, nh):
    xw = (jnp.dot(xe_ref[...], w1l_ref[...], preferred_element_type=jnp.float32)
          + jnp.dot(xo_ref[...], w1r_ref[...], preferred_element_type=jnp.float32))
    xw_ref[:nh, :] = xw
    xw_ref[nh:, :] = jnp.zeros_like(xw_ref[nh:, :])


def _tcb_body(xw1_ref, deg_ref, y1_ref):
    y1_ref[...] = xw1_ref[...] * _dinv_paired(deg_ref)


def _dinv_paired(deg_ref):
    d = deg_ref[...]
    dinv2 = lax.rsqrt(1.0 + d[0] + d[1])
    a = jnp.broadcast_to(dinv2[:, :1], (dinv2.shape[0], 64))
    b = jnp.broadcast_to(dinv2[:, 1:2], (dinv2.shape[0], 64))
    return jnp.concatenate([a, b], axis=1)


def _tc2_body(y1_ref, s_ref, deg_ref, b1_ref, w2_ref, y2_ref, *, nh):
    dp = _dinv_paired(deg_ref)
    s = s_ref[...]
    h = jnp.maximum(dp * (y1_ref[...] + s[0] + s[1]) + b1_ref[...], 0.0)
    rows = lax.broadcasted_iota(jnp.int32, h.shape, 0)
    h = jnp.where(rows < nh, h, 0.0)
    y2_ref[...] = jnp.dot(h, w2_ref[...], preferred_element_type=jnp.float32) * dp


def _tc3_body(y2_ref, s_ref, deg_ref, b2_ref, out_ref, *, nh):
    dp = _dinv_paired(deg_ref)
    s = s_ref[...]
    tot = dp * (y2_ref[...] + s[0] + s[1]) + b2_ref[...]
    out_ref[...] = tot[:nh, :]


def kernel(x, edge_index, W1, b1, W2, b2):
    n, fin = x.shape
    fh = W1.shape[1]
    e = edge_index.shape[1]
    f32 = jnp.float32

    npad = ((n + 16 + 511) // 512) * 512
    ch = -(-e // (_NW * _LN))
    ch = ((ch + 7) // 8) * 8
    epad = _NW * _LN * ch

    src = edge_index[0].astype(jnp.int32)
    dst = edge_index[1].astype(jnp.int32)
    pad_idx = n + (jnp.arange(epad - e, dtype=jnp.int32) % (npad - n))
    srcg = jnp.concatenate([src, pad_idx]).reshape(_NW, ch * _LN)
    dstg = jnp.concatenate([dst, pad_idx]).reshape(_NW, ch * _LN)

    x_even = x[0::2]
    x_odd = x[1::2]
    w1l = jnp.zeros((fin, 128), f32).at[:, :fh].set(W1)
    w1r = jnp.zeros((fin, 128), f32).at[:, fh:].set(W1)
    w2bd = (jnp.zeros((128, 128), f32)
            .at[:fh, :fh].set(W2).at[fh:, fh:].set(W2))
    b1p = jnp.concatenate([b1, b1]).reshape(1, 128)
    b2p = jnp.concatenate([b2, b2]).reshape(1, 128)

    nh = n // 2
    nph = npad // 2

    deg_parts = _make_deg_kernel(npad, ch)(dstg)
    deg2 = deg_parts.reshape(_NC, nph, 2)

    xw1 = pl.pallas_call(
        functools.partial(_tca_body, nh=nh),
        out_shape=jax.ShapeDtypeStruct((nph, 128), f32),
    )(x_even, x_odd, w1l, w1r)

    y1 = pl.pallas_call(
        _tcb_body,
        out_shape=jax.ShapeDtypeStruct((nph, 128), f32),
    )(xw1, deg2)

    msg = _make_msg_kernel(npad, ch)
    s1 = msg(y1.reshape(npad, 64), srcg, dstg)
    s1p = s1.reshape(_NC, nph, 128)

    y2 = pl.pallas_call(
        functools.partial(_tc2_body, nh=nh),
        out_shape=jax.ShapeDtypeStruct((nph, 128), f32),
    )(y1, s1p, deg2, b1p, w2bd)

    s2 = msg(y2.reshape(npad, 64), srcg, dstg)
    s2p = s2.reshape(_NC, nph, 128)

    out = pl.pallas_call(
        functools.partial(_tc3_body, nh=nh),
        out_shape=jax.ShapeDtypeStruct((nh, 128), f32),
    )(y2, s2p, deg2, b2p)
    return out.reshape(n, fh)
